# baseline (speedup 1.0000x reference)
import functools

import jax
import jax.numpy as jnp
from jax import lax
from jax.experimental import pallas as pl
from jax.experimental.pallas import tpu as pltpu
from jax.experimental.pallas import tpu_sc as plsc

B, S, H = 2, 2048, 2048
E = 8
I_MOE = 1408
I_SHARED = 5632
N = B * S

TB = 512
NB = N // TB
TM = 256
NBLK = 40
PADN = NBLK * TM
F_TILE = 1408
NF = I_SHARED // F_TILE

NC, NS = 2, 16
NW = NC * NS
TPW = N // NW
SUB = 16
NSUBC = TPW // SUB
CSUB = 8
NCSUB = TPW // CSUB


def _router_body(xf_ref, gw_ref, logits_ref, w1_ref, w2_ref, e1_ref, e2_ref,
                 r1_ref, r2_ref, bc_ref):
    x = xf_ref[...]
    logits = lax.dot_general(
        x.astype(jnp.bfloat16), gw_ref[...].astype(jnp.bfloat16),
        (((1,), (0,)), ((), ())),
        preferred_element_type=jnp.float32)
    logits_ref[...] = logits
    m = jnp.max(logits, axis=1, keepdims=True)
    ex = jnp.exp(logits - m)
    p = ex / jnp.sum(ex, axis=1, keepdims=True)
    iota = lax.broadcasted_iota(jnp.int32, p.shape, 1)
    m1 = jnp.max(p, axis=1, keepdims=True)
    i1 = jnp.min(jnp.where(p == m1, iota, E), axis=1, keepdims=True)
    p2 = jnp.where(iota == i1, -jnp.inf, p)
    m2 = jnp.max(p2, axis=1, keepdims=True)
    i2 = jnp.min(jnp.where(p2 == m2, iota, E), axis=1, keepdims=True)
    den = m1 + m2
    w1_ref[...] = jnp.broadcast_to(m1 / den, (TB, 128))
    w2_ref[...] = jnp.broadcast_to(m2 / den, (TB, 128))
    e1_ref[...] = i1[:, 0]
    e2_ref[...] = i2[:, 0]
    oh1 = (iota == i1).astype(jnp.float32)
    oh2 = (iota == i2).astype(jnp.float32)
    ohcat = jnp.concatenate([oh1, oh2], axis=0)
    rr = lax.broadcasted_iota(jnp.int32, (2 * TB, 2 * TB), 0)
    cc = lax.broadcasted_iota(jnp.int32, (2 * TB, 2 * TB), 1)
    ltri = (cc < rr).astype(jnp.float32)
    rank = lax.dot_general(ltri, ohcat, (((1,), (0,)), ((), ())),
                           preferred_element_type=jnp.float32)
    lr = jnp.sum(rank * ohcat, axis=1)
    r1_ref[...] = lr[:TB].astype(jnp.int32)
    r2_ref[...] = lr[TB:].astype(jnp.int32)
    bc_ref[0, 0, :] = jnp.sum(ohcat, axis=0)


def _router(xf, gate_w):
    return pl.pallas_call(
        _router_body,
        grid=(NB,),
        in_specs=[
            pl.BlockSpec((TB, H), lambda b: (b, 0)),
            pl.BlockSpec((H, E), lambda b: (0, 0)),
        ],
        out_specs=[
            pl.BlockSpec((TB, E), lambda b: (b, 0)),
            pl.BlockSpec((TB, 128), lambda b: (b, 0)),
            pl.BlockSpec((TB, 128), lambda b: (b, 0)),
            pl.BlockSpec((TB,), lambda b: (b,)),
            pl.BlockSpec((TB,), lambda b: (b,)),
            pl.BlockSpec((TB,), lambda b: (b,)),
            pl.BlockSpec((TB,), lambda b: (b,)),
            pl.BlockSpec((1, 1, E), lambda b: (b, 0, 0)),
        ],
        out_shape=[
            jax.ShapeDtypeStruct((N, E), jnp.float32),
            jax.ShapeDtypeStruct((N, 128), jnp.float32),
            jax.ShapeDtypeStruct((N, 128), jnp.float32),
            jax.ShapeDtypeStruct((N,), jnp.int32),
            jax.ShapeDtypeStruct((N,), jnp.int32),
            jax.ShapeDtypeStruct((N,), jnp.int32),
            jax.ShapeDtypeStruct((N,), jnp.int32),
            jax.ShapeDtypeStruct((NB, 1, E), jnp.float32),
        ],
        compiler_params=pltpu.CompilerParams(
            dimension_semantics=("arbitrary",)),
    )(xf, gate_w)


def _binning_body(e1_ref, e2_ref, r1_ref, r2_ref, bc_ref,
                  pos1_ref, pos2_ref, meta_ref):
    bc = bc_ref[...].reshape(NB, E)
    rr = lax.broadcasted_iota(jnp.int32, (NB, NB), 0)
    cc = lax.broadcasted_iota(jnp.int32, (NB, NB), 1)
    ltri = (cc < rr).astype(jnp.float32)
    bbase = lax.dot_general(ltri, bc, (((1,), (0,)), ((), ())),
                            preferred_element_type=jnp.float32)
    cnt = jnp.sum(bc, axis=0).astype(jnp.int32)
    padded = ((cnt + TM - 1) // TM) * TM
    rre = lax.broadcasted_iota(jnp.int32, (E, E), 0)
    cce = lax.broadcasted_iota(jnp.int32, (E, E), 1)
    ltrie = (cce < rre).astype(jnp.float32)
    seg = lax.dot_general(ltrie, padded.astype(jnp.float32)[:, None],
                          (((1,), (0,)), ((), ())),
                          preferred_element_type=jnp.float32)
    seg = seg[:, 0].astype(jnp.int32)
    nused = jnp.sum(padded) // TM

    jv = lax.iota(jnp.int32, 64)
    be = jnp.full((64,), -1, jnp.int32)
    for e in range(E):
        be = be + (jv * TM >= seg[e]).astype(jnp.int32)
    be = jnp.where(jv == 63, nused, be)
    meta_ref[0, :] = be

    tok = lax.iota(jnp.int32, N)
    blk = tok // TB
    bbase_i = bbase.astype(jnp.int32)
    for (e_ref, r_ref, pos_ref) in ((e1_ref, r1_ref, pos1_ref),
                                    (e2_ref, r2_ref, pos2_ref)):
        eid = e_ref[...]
        pos = r_ref[...]
        for e in range(E):
            pos = pos + jnp.where(eid == e, seg[e], 0)
            for b in range(NB):
                pos = pos + jnp.where((eid == e) & (blk == b),
                                      bbase_i[b, e], 0)
        pos_ref[...] = pos


def _binning(e1, e2, r1, r2, bc):
    return pl.pallas_call(
        _binning_body,
        grid=(1,),
        in_specs=[
            pl.BlockSpec((N,), lambda i: (0,)),
            pl.BlockSpec((N,), lambda i: (0,)),
            pl.BlockSpec((N,), lambda i: (0,)),
            pl.BlockSpec((N,), lambda i: (0,)),
            pl.BlockSpec((NB, 1, E), lambda i: (0, 0, 0)),
        ],
        out_specs=[
            pl.BlockSpec((N,), lambda i: (0,)),
            pl.BlockSpec((N,), lambda i: (0,)),
            pl.BlockSpec((1, 64), lambda i: (0, 0)),
        ],
        out_shape=[
            jax.ShapeDtypeStruct((N,), jnp.int32),
            jax.ShapeDtypeStruct((N,), jnp.int32),
            jax.ShapeDtypeStruct((1, 64), jnp.int32),
        ],
    )(e1, e2, r1, r2, bc)


def _mesh():
    return plsc.VectorSubcoreMesh(core_axis_name="c", subcore_axis_name="s",
                                  num_cores=NC, num_subcores=NS)


def _dispatch_body(hid, wbc1, wbc2, pos1, pos2, xs, ws,
                   posf, posv, rowbuf, wbuf, sems):
    c = lax.axis_index("c")
    s = lax.axis_index("s")
    wid = s * NC + c
    base = wid * TPW
    pltpu.sync_copy(pos1.at[pl.ds(base, TPW)], posf.at[0])
    pltpu.sync_copy(pos2.at[pl.ds(base, TPW)], posf.at[1])
    for k in range(2):
        for sub in range(NSUBC):
            for q in range(SUB // 16):
                v = posf[k, pl.ds(sub * SUB + q * 16, 16)]
                posv[k * NSUBC + sub, pl.ds(q * 16, 16)] = v
    cps = [None] * NSUBC
    for sub in range(NSUBC):
        t0 = base + sub * SUB
        d = sub % 2
        if cps[sub - 2] is not None:
            for cp in cps[sub - 2]:
                cp.wait()
            cps[sub - 2] = None
        pltpu.sync_copy(hid.at[pl.ds(t0, SUB)], rowbuf.at[d])
        pltpu.sync_copy(wbc1.at[pl.ds(t0, SUB)], wbuf.at[2 * d])
        pltpu.sync_copy(wbc2.at[pl.ds(t0, SUB)], wbuf.at[2 * d + 1])
        cps[sub] = [
            pltpu.async_copy(rowbuf.at[d], xs.at[posv.at[sub]], sems.at[d]),
            pltpu.async_copy(rowbuf.at[d], xs.at[posv.at[NSUBC + sub]],
                             sems.at[d]),
            pltpu.async_copy(wbuf.at[2 * d], ws.at[posv.at[sub]],
                             sems.at[2 + d]),
            pltpu.async_copy(wbuf.at[2 * d + 1], ws.at[posv.at[NSUBC + sub]],
                             sems.at[2 + d]),
        ]
    for cpl in cps:
        if cpl is not None:
            for cp in cpl:
                cp.wait()


def _dispatch(xf, wbc1, wbc2, pos1, pos2):
    return pl.kernel(
        _dispatch_body,
        out_type=[
            jax.ShapeDtypeStruct((PADN, H), jnp.float32),
            jax.ShapeDtypeStruct((PADN, 128), jnp.float32),
        ],
        mesh=_mesh(),
        scratch_types=[
            pltpu.VMEM((2, TPW), jnp.int32),
            pltpu.VMEM((2 * NSUBC, SUB), jnp.int32),
            pltpu.VMEM((2, SUB, H), jnp.float32),
            pltpu.VMEM((4, SUB, 128), jnp.float32),
            pltpu.SemaphoreType.DMA((4,)),
        ],
    )(xf, wbc1, wbc2, pos1, pos2)


def _grouped_body(meta_ref, xs_ref, ws_ref, wg_ref, wu_ref, wd_ref, ys_ref):
    j = pl.program_id(0)

    @pl.when(j < meta_ref[0, 63])
    def _():
        x = xs_ref[...].astype(jnp.bfloat16)
        g = jnp.dot(x, wg_ref[0], preferred_element_type=jnp.float32)
        u = jnp.dot(x, wu_ref[0], preferred_element_type=jnp.float32)
        h = (g * jax.nn.sigmoid(g) * u).astype(jnp.bfloat16)
        y = jnp.dot(h, wd_ref[0], preferred_element_type=jnp.float32)
        ys_ref[...] = y * ws_ref[:, 0:1]


def _grouped(xs, ws, wg, wu, wd, meta):
    grid_spec = pltpu.PrefetchScalarGridSpec(
        num_scalar_prefetch=1,
        grid=(NBLK,),
        in_specs=[
            pl.BlockSpec((TM, H), lambda j, m: (j, 0)),
            pl.BlockSpec((TM, 128), lambda j, m: (j, 0)),
            pl.BlockSpec((1, H, I_MOE), lambda j, m: (m[0, j], 0, 0)),
            pl.BlockSpec((1, H, I_MOE), lambda j, m: (m[0, j], 0, 0)),
            pl.BlockSpec((1, I_MOE, H), lambda j, m: (m[0, j], 0, 0)),
        ],
        out_specs=pl.BlockSpec((TM, H), lambda j, m: (j, 0)),
    )
    return pl.pallas_call(
        _grouped_body,
        grid_spec=grid_spec,
        out_shape=jax.ShapeDtypeStruct((PADN, H), jnp.float32),
        compiler_params=pltpu.CompilerParams(
            dimension_semantics=("arbitrary",)),
    )(meta, xs, ws, wg, wu, wd)


def _shared_body(xb_ref, sg_ref, su_ref, sd_ref, sgw_ref, out_ref):
    f = pl.program_id(1)

    @pl.when(f == 0)
    def _init():
        out_ref[...] = jnp.zeros_like(out_ref)

    x = xb_ref[...]
    g = jnp.dot(x, sg_ref[...], preferred_element_type=jnp.float32)
    u = jnp.dot(x, su_ref[...], preferred_element_type=jnp.float32)
    h = (g * jax.nn.sigmoid(g) * u).astype(jnp.bfloat16)
    out_ref[...] += jnp.dot(h, sd_ref[...], preferred_element_type=jnp.float32)

    @pl.when(f == NF - 1)
    def _finish():
        gate = jnp.dot(x, sgw_ref[...], preferred_element_type=jnp.float32)
        out_ref[...] *= jax.nn.sigmoid(gate)


def _shared(xb, sg, su, sd, sgw):
    return pl.pallas_call(
        _shared_body,
        grid=(NB, NF),
        in_specs=[
            pl.BlockSpec((TB, H), lambda b, f: (b, 0)),
            pl.BlockSpec((H, F_TILE), lambda b, f: (0, f)),
            pl.BlockSpec((H, F_TILE), lambda b, f: (0, f)),
            pl.BlockSpec((F_TILE, H), lambda b, f: (f, 0)),
            pl.BlockSpec((H, 1), lambda b, f: (0, 0)),
        ],
        out_specs=pl.BlockSpec((TB, H), lambda b, f: (b, 0)),
        out_shape=jax.ShapeDtypeStruct((N, H), jnp.float32),
        compiler_params=pltpu.CompilerParams(
            dimension_semantics=("arbitrary", "arbitrary")),
    )(xb, sg, su, sd, sgw)


def _combine_body(ys, pos1, pos2, out0, out1, posf, posv, buf, buf2, sems):
    c = lax.axis_index("c")
    s = lax.axis_index("s")
    wid = s * NC + c
    base = wid * TPW
    pltpu.sync_copy(pos1.at[pl.ds(base, TPW)], posf.at[0])
    pltpu.sync_copy(pos2.at[pl.ds(base, TPW)], posf.at[1])
    for k in range(2):
        for sub in range(NCSUB):
            v = posf[k, pl.ds(sub * CSUB, CSUB)]
            posv[k * NCSUB + sub, :] = v
    gat = [None] * NCSUB
    wrt = [None] * NCSUB

    def fire_gather(sub):
        d = sub % 2
        gat[sub] = [
            pltpu.async_copy(ys.at[posv.at[sub]], buf.at[d], sems.at[0]),
            pltpu.async_copy(ys.at[posv.at[NCSUB + sub]], buf2.at[d],
                             sems.at[1]),
        ]

    def fire_write(sub):
        t0 = base + sub * CSUB
        d = sub % 2
        for cp in gat[sub]:
            cp.wait()
        wrt[sub] = [
            pltpu.async_copy(buf.at[d], out0.at[pl.ds(t0, CSUB)], sems.at[2]),
            pltpu.async_copy(buf2.at[d], out1.at[pl.ds(t0, CSUB)],
                             sems.at[3]),
        ]

    for sub in range(NCSUB):
        if sub >= 2:
            for cp in wrt[sub - 2]:
                cp.wait()
            wrt[sub - 2] = None
        fire_gather(sub)
        if sub >= 1:
            fire_write(sub - 1)
    fire_write(NCSUB - 1)
    for cpl in wrt:
        if cpl is not None:
            for cp in cpl:
                cp.wait()


def _combine(ys, pos1, pos2):
    return pl.kernel(
        _combine_body,
        out_type=[
            jax.ShapeDtypeStruct((N, H), jnp.float32),
            jax.ShapeDtypeStruct((N, H), jnp.float32),
        ],
        mesh=_mesh(),
        scratch_types=[
            pltpu.VMEM((2, TPW), jnp.int32),
            pltpu.VMEM((2 * NCSUB, CSUB), jnp.int32),
            pltpu.VMEM((2, CSUB, H), jnp.float32),
            pltpu.VMEM((2, CSUB, H), jnp.float32),
            pltpu.SemaphoreType.DMA((4,)),
        ],
    )(ys, pos1, pos2)


def _final_body(y0_ref, y1_ref, shr_ref, out_ref):
    out_ref[...] = y0_ref[...] + y1_ref[...] + shr_ref[...]


def _final(y0, y1, shr):
    return pl.pallas_call(
        _final_body,
        grid=(NB,),
        in_specs=[
            pl.BlockSpec((TB, H), lambda b: (b, 0)),
            pl.BlockSpec((TB, H), lambda b: (b, 0)),
            pl.BlockSpec((TB, H), lambda b: (b, 0)),
        ],
        out_specs=pl.BlockSpec((TB, H), lambda b: (b, 0)),
        out_shape=jax.ShapeDtypeStruct((N, H), jnp.float32),
        compiler_params=pltpu.CompilerParams(
            dimension_semantics=("arbitrary",)),
    )(y0, y1, shr)


def kernel(hidden_states, gate_w, expert_gate, expert_up, expert_down,
           shared_gate, shared_up, shared_down, shared_expert_gate_w):
    xf = hidden_states.reshape(N, H)
    xb = xf.astype(jnp.bfloat16)
    wg = expert_gate.astype(jnp.bfloat16)
    wu = expert_up.astype(jnp.bfloat16)
    wd = expert_down.astype(jnp.bfloat16)
    sg = shared_gate.astype(jnp.bfloat16)
    su = shared_up.astype(jnp.bfloat16)
    sd = shared_down.astype(jnp.bfloat16)
    sgw = shared_expert_gate_w.astype(jnp.bfloat16)

    logits, wbc1, wbc2, e1, e2, r1, r2, bc = _router(xf, gate_w)
    pos1, pos2, meta = _binning(e1, e2, r1, r2, bc)
    xs, ws = _dispatch(xf, wbc1, wbc2, pos1, pos2)
    shared_out = _shared(xb, sg, su, sd, sgw)
    ys = _grouped(xs, ws, wg, wu, wd, meta)
    y0, y1 = _combine(ys, pos1, pos2)
    final = _final(y0, y1, shared_out)
    return final.reshape(B, S, H), logits.reshape(B, S, E)


# speedup vs baseline: 1.0419x; 1.0419x over previous
import functools

import jax
import jax.numpy as jnp
from jax import lax
from jax.experimental import pallas as pl
from jax.experimental.pallas import tpu as pltpu
from jax.experimental.pallas import tpu_sc as plsc

B, S, H = 2, 2048, 2048
E = 8
I_MOE = 1408
I_SHARED = 5632
N = B * S

TB = 512
NB = N // TB
TM = 512
NBLK = 23
PADN = NBLK * TM
F_TILE = 512
NF = I_SHARED // F_TILE
TBS = 1024

NC, NS = 2, 16
NW = NC * NS
TPW = N // NW
SUB = 16
NSUBC = TPW // SUB
CSUB = 8
NCSUB = TPW // CSUB


def _router_body(xf_ref, gw_ref, logits_ref, w1_ref, w2_ref, e1_ref, e2_ref,
                 r1_ref, r2_ref, bc_ref):
    x = xf_ref[...]
    logits = lax.dot_general(
        x.astype(jnp.bfloat16), gw_ref[...].astype(jnp.bfloat16),
        (((1,), (0,)), ((), ())),
        preferred_element_type=jnp.float32)
    logits_ref[...] = logits
    m = jnp.max(logits, axis=1, keepdims=True)
    ex = jnp.exp(logits - m)
    p = ex / jnp.sum(ex, axis=1, keepdims=True)
    iota = lax.broadcasted_iota(jnp.int32, p.shape, 1)
    m1 = jnp.max(p, axis=1, keepdims=True)
    i1 = jnp.min(jnp.where(p == m1, iota, E), axis=1, keepdims=True)
    p2 = jnp.where(iota == i1, -jnp.inf, p)
    m2 = jnp.max(p2, axis=1, keepdims=True)
    i2 = jnp.min(jnp.where(p2 == m2, iota, E), axis=1, keepdims=True)
    den = m1 + m2
    w1_ref[...] = jnp.broadcast_to(m1 / den, (TB, 128))
    w2_ref[...] = jnp.broadcast_to(m2 / den, (TB, 128))
    e1_ref[...] = i1[:, 0]
    e2_ref[...] = i2[:, 0]
    oh1 = (iota == i1).astype(jnp.float32)
    oh2 = (iota == i2).astype(jnp.float32)
    ohcat = jnp.concatenate([oh1, oh2], axis=0)
    rr = lax.broadcasted_iota(jnp.int32, (2 * TB, 2 * TB), 0)
    cc = lax.broadcasted_iota(jnp.int32, (2 * TB, 2 * TB), 1)
    ltri = (cc < rr).astype(jnp.float32)
    rank = lax.dot_general(ltri, ohcat, (((1,), (0,)), ((), ())),
                           preferred_element_type=jnp.float32)
    lr = jnp.sum(rank * ohcat, axis=1)
    r1_ref[...] = lr[:TB].astype(jnp.int32)
    r2_ref[...] = lr[TB:].astype(jnp.int32)
    bc_ref[0, 0, :] = jnp.sum(ohcat, axis=0)


def _router(xf, gate_w):
    return pl.pallas_call(
        _router_body,
        grid=(NB,),
        in_specs=[
            pl.BlockSpec((TB, H), lambda b: (b, 0)),
            pl.BlockSpec((H, E), lambda b: (0, 0)),
        ],
        out_specs=[
            pl.BlockSpec((TB, E), lambda b: (b, 0)),
            pl.BlockSpec((TB, 128), lambda b: (b, 0)),
            pl.BlockSpec((TB, 128), lambda b: (b, 0)),
            pl.BlockSpec((TB,), lambda b: (b,)),
            pl.BlockSpec((TB,), lambda b: (b,)),
            pl.BlockSpec((TB,), lambda b: (b,)),
            pl.BlockSpec((TB,), lambda b: (b,)),
            pl.BlockSpec((1, 1, E), lambda b: (b, 0, 0)),
        ],
        out_shape=[
            jax.ShapeDtypeStruct((N, E), jnp.float32),
            jax.ShapeDtypeStruct((N, 128), jnp.float32),
            jax.ShapeDtypeStruct((N, 128), jnp.float32),
            jax.ShapeDtypeStruct((N,), jnp.int32),
            jax.ShapeDtypeStruct((N,), jnp.int32),
            jax.ShapeDtypeStruct((N,), jnp.int32),
            jax.ShapeDtypeStruct((N,), jnp.int32),
            jax.ShapeDtypeStruct((NB, 1, E), jnp.float32),
        ],
        compiler_params=pltpu.CompilerParams(
            dimension_semantics=("arbitrary",)),
    )(xf, gate_w)


def _binning_body(e1_ref, e2_ref, r1_ref, r2_ref, bc_ref,
                  pos1_ref, pos2_ref, meta_ref):
    bc = bc_ref[...].reshape(NB, E)
    rr = lax.broadcasted_iota(jnp.int32, (NB, NB), 0)
    cc = lax.broadcasted_iota(jnp.int32, (NB, NB), 1)
    ltri = (cc < rr).astype(jnp.float32)
    bbase = lax.dot_general(ltri, bc, (((1,), (0,)), ((), ())),
                            preferred_element_type=jnp.float32)
    cnt = jnp.sum(bc, axis=0).astype(jnp.int32)
    padded = ((cnt + TM - 1) // TM) * TM
    rre = lax.broadcasted_iota(jnp.int32, (E, E), 0)
    cce = lax.broadcasted_iota(jnp.int32, (E, E), 1)
    ltrie = (cce < rre).astype(jnp.float32)
    seg = lax.dot_general(ltrie, padded.astype(jnp.float32)[:, None],
                          (((1,), (0,)), ((), ())),
                          preferred_element_type=jnp.float32)
    seg = seg[:, 0].astype(jnp.int32)
    nused = jnp.sum(padded) // TM

    jv = lax.iota(jnp.int32, 64)
    be = jnp.full((64,), -1, jnp.int32)
    for e in range(E):
        be = be + (jv * TM >= seg[e]).astype(jnp.int32)
    be = jnp.where(jv == 63, nused, be)
    meta_ref[0, :] = be

    tok = lax.iota(jnp.int32, N)
    blk = tok // TB
    bbase_i = bbase.astype(jnp.int32)
    for (e_ref, r_ref, pos_ref) in ((e1_ref, r1_ref, pos1_ref),
                                    (e2_ref, r2_ref, pos2_ref)):
        eid = e_ref[...]
        pos = r_ref[...]
        for e in range(E):
            pos = pos + jnp.where(eid == e, seg[e], 0)
            for b in range(NB):
                pos = pos + jnp.where((eid == e) & (blk == b),
                                      bbase_i[b, e], 0)
        pos_ref[...] = pos


def _binning(e1, e2, r1, r2, bc):
    return pl.pallas_call(
        _binning_body,
        grid=(1,),
        in_specs=[
            pl.BlockSpec((N,), lambda i: (0,)),
            pl.BlockSpec((N,), lambda i: (0,)),
            pl.BlockSpec((N,), lambda i: (0,)),
            pl.BlockSpec((N,), lambda i: (0,)),
            pl.BlockSpec((NB, 1, E), lambda i: (0, 0, 0)),
        ],
        out_specs=[
            pl.BlockSpec((N,), lambda i: (0,)),
            pl.BlockSpec((N,), lambda i: (0,)),
            pl.BlockSpec((1, 64), lambda i: (0, 0)),
        ],
        out_shape=[
            jax.ShapeDtypeStruct((N,), jnp.int32),
            jax.ShapeDtypeStruct((N,), jnp.int32),
            jax.ShapeDtypeStruct((1, 64), jnp.int32),
        ],
    )(e1, e2, r1, r2, bc)


def _mesh():
    return plsc.VectorSubcoreMesh(core_axis_name="c", subcore_axis_name="s",
                                  num_cores=NC, num_subcores=NS)


def _dispatch_body(hid, wbc1, wbc2, pos1, pos2, xs, ws,
                   posf, posv, rowbuf, wbuf, sems):
    c = lax.axis_index("c")
    s = lax.axis_index("s")
    wid = s * NC + c
    base = wid * TPW
    pltpu.sync_copy(pos1.at[pl.ds(base, TPW)], posf.at[0])
    pltpu.sync_copy(pos2.at[pl.ds(base, TPW)], posf.at[1])
    for k in range(2):
        for sub in range(NSUBC):
            for q in range(SUB // 16):
                v = posf[k, pl.ds(sub * SUB + q * 16, 16)]
                posv[k * NSUBC + sub, pl.ds(q * 16, 16)] = v
    cps = [None] * NSUBC
    for sub in range(NSUBC):
        t0 = base + sub * SUB
        d = sub % 2
        if cps[sub - 2] is not None:
            for cp in cps[sub - 2]:
                cp.wait()
            cps[sub - 2] = None
        pltpu.sync_copy(hid.at[pl.ds(t0, SUB)], rowbuf.at[d])
        pltpu.sync_copy(wbc1.at[pl.ds(t0, SUB)], wbuf.at[2 * d])
        pltpu.sync_copy(wbc2.at[pl.ds(t0, SUB)], wbuf.at[2 * d + 1])
        cps[sub] = [
            pltpu.async_copy(rowbuf.at[d], xs.at[posv.at[sub]], sems.at[d]),
            pltpu.async_copy(rowbuf.at[d], xs.at[posv.at[NSUBC + sub]],
                             sems.at[d]),
            pltpu.async_copy(wbuf.at[2 * d], ws.at[posv.at[sub]],
                             sems.at[2 + d]),
            pltpu.async_copy(wbuf.at[2 * d + 1], ws.at[posv.at[NSUBC + sub]],
                             sems.at[2 + d]),
        ]
    for cpl in cps:
        if cpl is not None:
            for cp in cpl:
                cp.wait()


def _dispatch(xf, wbc1, wbc2, pos1, pos2):
    return pl.kernel(
        _dispatch_body,
        out_type=[
            jax.ShapeDtypeStruct((PADN, H), jnp.float32),
            jax.ShapeDtypeStruct((PADN, 128), jnp.float32),
        ],
        mesh=_mesh(),
        scratch_types=[
            pltpu.VMEM((2, TPW), jnp.int32),
            pltpu.VMEM((2 * NSUBC, SUB), jnp.int32),
            pltpu.VMEM((2, SUB, H), jnp.float32),
            pltpu.VMEM((4, SUB, 128), jnp.float32),
            pltpu.SemaphoreType.DMA((4,)),
        ],
    )(xf, wbc1, wbc2, pos1, pos2)


def _grouped_body(meta_ref, xs_ref, ws_ref, wg_ref, wu_ref, wd_ref, ys_ref):
    j = pl.program_id(0)

    @pl.when(j < meta_ref[0, 63])
    def _():
        x = xs_ref[...].astype(jnp.bfloat16)
        g = jnp.dot(x, wg_ref[0], preferred_element_type=jnp.float32)
        u = jnp.dot(x, wu_ref[0], preferred_element_type=jnp.float32)
        h = (g * jax.nn.sigmoid(g) * u).astype(jnp.bfloat16)
        y = jnp.dot(h, wd_ref[0], preferred_element_type=jnp.float32)
        ys_ref[...] = y * ws_ref[:, 0:1]


def _grouped(xs, ws, wg, wu, wd, meta):
    grid_spec = pltpu.PrefetchScalarGridSpec(
        num_scalar_prefetch=1,
        grid=(NBLK,),
        in_specs=[
            pl.BlockSpec((TM, H), lambda j, m: (j, 0)),
            pl.BlockSpec((TM, 128), lambda j, m: (j, 0)),
            pl.BlockSpec((1, H, I_MOE), lambda j, m: (m[0, j], 0, 0)),
            pl.BlockSpec((1, H, I_MOE), lambda j, m: (m[0, j], 0, 0)),
            pl.BlockSpec((1, I_MOE, H), lambda j, m: (m[0, j], 0, 0)),
        ],
        out_specs=pl.BlockSpec((TM, H), lambda j, m: (j, 0)),
    )
    return pl.pallas_call(
        _grouped_body,
        grid_spec=grid_spec,
        out_shape=jax.ShapeDtypeStruct((PADN, H), jnp.float32),
        compiler_params=pltpu.CompilerParams(
            dimension_semantics=("arbitrary",)),
    )(meta, xs, ws, wg, wu, wd)


def _shared_body(xb_ref, sg_ref, su_ref, sd_ref, sgw_ref, out_ref):
    f = pl.program_id(1)

    @pl.when(f == 0)
    def _init():
        out_ref[...] = jnp.zeros_like(out_ref)

    x = xb_ref[...].astype(jnp.bfloat16)
    g = jnp.dot(x, sg_ref[...], preferred_element_type=jnp.float32)
    u = jnp.dot(x, su_ref[...], preferred_element_type=jnp.float32)
    h = (g * jax.nn.sigmoid(g) * u).astype(jnp.bfloat16)
    out_ref[...] += jnp.dot(h, sd_ref[...], preferred_element_type=jnp.float32)

    @pl.when(f == NF - 1)
    def _finish():
        gate = jnp.dot(x, sgw_ref[...].astype(jnp.bfloat16),
                       preferred_element_type=jnp.float32)
        out_ref[...] *= jax.nn.sigmoid(gate)


def _shared(xb, sg, su, sd, sgw):
    return pl.pallas_call(
        _shared_body,
        grid=(N // TBS, NF),
        in_specs=[
            pl.BlockSpec((TBS, H), lambda b, f: (b, 0)),
            pl.BlockSpec((H, F_TILE), lambda b, f: (0, f)),
            pl.BlockSpec((H, F_TILE), lambda b, f: (0, f)),
            pl.BlockSpec((F_TILE, H), lambda b, f: (f, 0)),
            pl.BlockSpec((H, 1), lambda b, f: (0, 0)),
        ],
        out_specs=pl.BlockSpec((TBS, H), lambda b, f: (b, 0)),
        out_shape=jax.ShapeDtypeStruct((N, H), jnp.float32),
        compiler_params=pltpu.CompilerParams(
            dimension_semantics=("arbitrary", "arbitrary")),
    )(xb, sg, su, sd, sgw)


def _combine_body(ys, pos1, pos2, out0, out1, posf, posv, buf, buf2, sems):
    c = lax.axis_index("c")
    s = lax.axis_index("s")
    wid = s * NC + c
    base = wid * TPW
    pltpu.sync_copy(pos1.at[pl.ds(base, TPW)], posf.at[0])
    pltpu.sync_copy(pos2.at[pl.ds(base, TPW)], posf.at[1])
    for k in range(2):
        for sub in range(NCSUB):
            v = posf[k, pl.ds(sub * CSUB, CSUB)]
            posv[k * NCSUB + sub, :] = v
    gat = [None] * NCSUB
    wrt = [None] * NCSUB

    def fire_gather(sub):
        d = sub % 2
        gat[sub] = [
            pltpu.async_copy(ys.at[posv.at[sub]], buf.at[d], sems.at[0]),
            pltpu.async_copy(ys.at[posv.at[NCSUB + sub]], buf2.at[d],
                             sems.at[1]),
        ]

    def fire_write(sub):
        t0 = base + sub * CSUB
        d = sub % 2
        for cp in gat[sub]:
            cp.wait()
        wrt[sub] = [
            pltpu.async_copy(buf.at[d], out0.at[pl.ds(t0, CSUB)], sems.at[2]),
            pltpu.async_copy(buf2.at[d], out1.at[pl.ds(t0, CSUB)],
                             sems.at[3]),
        ]

    for sub in range(NCSUB):
        if sub >= 2:
            for cp in wrt[sub - 2]:
                cp.wait()
            wrt[sub - 2] = None
        fire_gather(sub)
        if sub >= 1:
            fire_write(sub - 1)
    fire_write(NCSUB - 1)
    for cpl in wrt:
        if cpl is not None:
            for cp in cpl:
                cp.wait()


def _combine(ys, pos1, pos2):
    return pl.kernel(
        _combine_body,
        out_type=[
            jax.ShapeDtypeStruct((N, H), jnp.float32),
            jax.ShapeDtypeStruct((N, H), jnp.float32),
        ],
        mesh=_mesh(),
        scratch_types=[
            pltpu.VMEM((2, TPW), jnp.int32),
            pltpu.VMEM((2 * NCSUB, CSUB), jnp.int32),
            pltpu.VMEM((2, CSUB, H), jnp.float32),
            pltpu.VMEM((2, CSUB, H), jnp.float32),
            pltpu.SemaphoreType.DMA((4,)),
        ],
    )(ys, pos1, pos2)


def _final_body(y0_ref, y1_ref, shr_ref, out_ref):
    out_ref[...] = y0_ref[...] + y1_ref[...] + shr_ref[...]


def _final(y0, y1, shr):
    return pl.pallas_call(
        _final_body,
        grid=(NB,),
        in_specs=[
            pl.BlockSpec((TB, H), lambda b: (b, 0)),
            pl.BlockSpec((TB, H), lambda b: (b, 0)),
            pl.BlockSpec((TB, H), lambda b: (b, 0)),
        ],
        out_specs=pl.BlockSpec((TB, H), lambda b: (b, 0)),
        out_shape=jax.ShapeDtypeStruct((N, H), jnp.float32),
        compiler_params=pltpu.CompilerParams(
            dimension_semantics=("arbitrary",)),
    )(y0, y1, shr)


def kernel(hidden_states, gate_w, expert_gate, expert_up, expert_down,
           shared_gate, shared_up, shared_down, shared_expert_gate_w):
    xf = hidden_states.reshape(N, H)
    wg = expert_gate.astype(jnp.bfloat16)
    wu = expert_up.astype(jnp.bfloat16)
    wd = expert_down.astype(jnp.bfloat16)
    sg = shared_gate.astype(jnp.bfloat16)
    su = shared_up.astype(jnp.bfloat16)
    sd = shared_down.astype(jnp.bfloat16)
    sgw = shared_expert_gate_w

    logits, wbc1, wbc2, e1, e2, r1, r2, bc = _router(xf, gate_w)
    pos1, pos2, meta = _binning(e1, e2, r1, r2, bc)
    xs, ws = _dispatch(xf, wbc1, wbc2, pos1, pos2)
    shared_out = _shared(xf, sg, su, sd, sgw)
    ys = _grouped(xs, ws, wg, wu, wd, meta)
    y0, y1 = _combine(ys, pos1, pos2)
    final = _final(y0, y1, shared_out)
    return final.reshape(B, S, H), logits.reshape(B, S, E)
